# initial kernel scaffold (unmeasured)
import jax
import jax.numpy as jnp
from jax import lax
from jax.experimental import pallas as pl
from jax.experimental.pallas import tpu as pltpu

NDEV = 16
_MESH = pl.DeviceIdType.MESH


def _body(x_ref, w_ref, out_ref, send_buf, recv_buf, out_sems,
          rs_send, rs_recv, ag_send, ag_recv, rs_credit, ag_credit):
    my = lax.axis_index("i")
    left = lax.rem(my + NDEV - 1, NDEV)
    right = lax.rem(my + 1, NDEV)

    m = x_ref.shape[0]
    ch = m // NDEV

    barrier = pltpu.get_barrier_semaphore()
    pl.semaphore_signal(barrier, inc=1, device_id=(left,), device_id_type=_MESH)
    pl.semaphore_signal(barrier, inc=1, device_id=(right,), device_id_type=_MESH)
    pl.semaphore_wait(barrier, 2)

    def cidx(off):
        return lax.rem(my + off + 2 * NDEV, NDEV)

    def chunk(c):
        return jnp.dot(x_ref[pl.ds(c * ch, ch), :], w_ref[...],
                       preferred_element_type=jnp.float32)

    def credit(sem):
        pl.semaphore_signal(sem, inc=1, device_id=(left,), device_id_type=_MESH)

    send_buf[0] = chunk(cidx(0))
    for s in range(NDEV - 1):
        slot = s % 2
        rdma = pltpu.make_async_remote_copy(
            src_ref=send_buf.at[slot],
            dst_ref=recv_buf.at[slot],
            send_sem=rs_send.at[s],
            recv_sem=rs_recv.at[s],
            device_id=(right,),
            device_id_type=_MESH,
        )
        if s >= 2:
            pl.semaphore_wait(rs_credit, 1)
        rdma.start()
        nxt = chunk(cidx(-s - 1))
        rdma.wait()
        acc = nxt + recv_buf[slot]
        if s < NDEV - 2:
            send_buf[1 - slot] = acc
            if s <= NDEV - 4:
                credit(rs_credit)
        else:
            send_buf[0] = acc
            credit(ag_credit)

    own = pltpu.make_async_copy(
        send_buf.at[0], out_ref.at[pl.ds(cidx(1) * ch, ch)], out_sems.at[0])
    own.start()
    own.wait()

    for t in range(NDEV - 1):
        b = t % 2
        src = send_buf.at[0] if t == 0 else recv_buf.at[1 - b]
        rdma = pltpu.make_async_remote_copy(
            src_ref=src,
            dst_ref=recv_buf.at[b],
            send_sem=ag_send.at[t],
            recv_sem=ag_recv.at[t],
            device_id=(right,),
            device_id_type=_MESH,
        )
        if t == 0 or t >= 2:
            pl.semaphore_wait(ag_credit, 1)
        rdma.start()
        rdma.wait()
        cp = pltpu.make_async_copy(
            recv_buf.at[b], out_ref.at[pl.ds(cidx(-t) * ch, ch)],
            out_sems.at[1])
        cp.start()
        cp.wait()
        if 1 <= t <= NDEV - 3:
            credit(ag_credit)


def _gemm_allreduce(x, w):
    m = x.shape[0]
    n = w.shape[1]
    ch = m // NDEV
    return pl.pallas_call(
        _body,
        out_shape=jax.ShapeDtypeStruct((m, n), jnp.float32),
        in_specs=[pl.BlockSpec(memory_space=pltpu.VMEM),
                  pl.BlockSpec(memory_space=pltpu.VMEM)],
        out_specs=pl.BlockSpec(memory_space=pltpu.ANY),
        scratch_shapes=[
            pltpu.VMEM((2, ch, n), jnp.float32),
            pltpu.VMEM((2, ch, n), jnp.float32),
            pltpu.SemaphoreType.DMA((2,)),
            pltpu.SemaphoreType.DMA((NDEV - 1,)),
            pltpu.SemaphoreType.DMA((NDEV - 1,)),
            pltpu.SemaphoreType.DMA((NDEV - 1,)),
            pltpu.SemaphoreType.DMA((NDEV - 1,)),
            pltpu.SemaphoreType.REGULAR,
            pltpu.SemaphoreType.REGULAR,
        ],
        compiler_params=pltpu.CompilerParams(collective_id=0),
    )(x, w)


def kernel(x, w_mat):
    y = _gemm_allreduce(x, w_mat)
    amax = jnp.max(jnp.abs(y))
    scale = amax / 448.0
    q = (y / scale).astype(jnp.float8_e4m3fn)
    return q.astype(jnp.float32) * scale


# baseline (device time: 3131407 ns/iter reference)
import jax
import jax.numpy as jnp
from jax import lax
from jax.experimental import pallas as pl
from jax.experimental.pallas import tpu as pltpu

NDEV = 16
_MESH = pl.DeviceIdType.MESH


def _body(x_ref, w_ref, out_ref, send_buf, recv_buf, out_sems,
          rs_send, rs_recv, ag_send, ag_recv, rs_credit, ag_credit):
    my = lax.axis_index("i")
    left = lax.rem(my + NDEV - 1, NDEV)
    right = lax.rem(my + 1, NDEV)

    m = x_ref.shape[0]
    ch = m // NDEV

    barrier = pltpu.get_barrier_semaphore()
    pl.semaphore_signal(barrier, inc=1, device_id=(left,), device_id_type=_MESH)
    pl.semaphore_signal(barrier, inc=1, device_id=(right,), device_id_type=_MESH)
    pl.semaphore_wait(barrier, 2)

    def cidx(off):
        return lax.rem(my + off + 2 * NDEV, NDEV)

    def chunk(c):
        return jnp.dot(x_ref[pl.ds(c * ch, ch), :], w_ref[...],
                       precision=lax.Precision.HIGHEST,
                       preferred_element_type=jnp.float32)

    def credit(sem):
        pl.semaphore_signal(sem, inc=1, device_id=(left,), device_id_type=_MESH)

    send_buf[0] = chunk(cidx(0))
    for s in range(NDEV - 1):
        slot = s % 2
        rdma = pltpu.make_async_remote_copy(
            src_ref=send_buf.at[slot],
            dst_ref=recv_buf.at[slot],
            send_sem=rs_send.at[s],
            recv_sem=rs_recv.at[s],
            device_id=(right,),
            device_id_type=_MESH,
        )
        if s >= 2:
            pl.semaphore_wait(rs_credit, 1)
        rdma.start()
        nxt = chunk(cidx(-s - 1))
        rdma.wait()
        acc = nxt + recv_buf[slot]
        if s < NDEV - 2:
            send_buf[1 - slot] = acc
            if s <= NDEV - 4:
                credit(rs_credit)
        else:
            send_buf[0] = acc
            credit(ag_credit)

    own = pltpu.make_async_copy(
        send_buf.at[0], out_ref.at[pl.ds(cidx(1) * ch, ch)], out_sems.at[0])
    own.start()
    own.wait()

    for t in range(NDEV - 1):
        b = t % 2
        src = send_buf.at[0] if t == 0 else recv_buf.at[1 - b]
        rdma = pltpu.make_async_remote_copy(
            src_ref=src,
            dst_ref=recv_buf.at[b],
            send_sem=ag_send.at[t],
            recv_sem=ag_recv.at[t],
            device_id=(right,),
            device_id_type=_MESH,
        )
        if t == 0 or t >= 2:
            pl.semaphore_wait(ag_credit, 1)
        rdma.start()
        rdma.wait()
        cp = pltpu.make_async_copy(
            recv_buf.at[b], out_ref.at[pl.ds(cidx(-t) * ch, ch)],
            out_sems.at[1])
        cp.start()
        cp.wait()
        if 1 <= t <= NDEV - 3:
            credit(ag_credit)


def _gemm_allreduce(x, w):
    m = x.shape[0]
    n = w.shape[1]
    ch = m // NDEV
    return pl.pallas_call(
        _body,
        out_shape=jax.ShapeDtypeStruct((m, n), jnp.float32),
        in_specs=[pl.BlockSpec(memory_space=pltpu.VMEM),
                  pl.BlockSpec(memory_space=pltpu.VMEM)],
        out_specs=pl.BlockSpec(memory_space=pl.ANY),
        scratch_shapes=[
            pltpu.VMEM((2, ch, n), jnp.float32),
            pltpu.VMEM((2, ch, n), jnp.float32),
            pltpu.SemaphoreType.DMA((2,)),
            pltpu.SemaphoreType.DMA((NDEV - 1,)),
            pltpu.SemaphoreType.DMA((NDEV - 1,)),
            pltpu.SemaphoreType.DMA((NDEV - 1,)),
            pltpu.SemaphoreType.DMA((NDEV - 1,)),
            pltpu.SemaphoreType.REGULAR,
            pltpu.SemaphoreType.REGULAR,
        ],
        compiler_params=pltpu.CompilerParams(
            collective_id=0,
            vmem_limit_bytes=100 * 1024 * 1024,
        ),
    )(x, w)


def kernel(x, w_mat):
    y = _gemm_allreduce(x, w_mat)
    amax = jnp.max(jnp.abs(y))
    scale = amax / 448.0
    q = (y / scale).astype(jnp.float8_e4m3fn)
    q = lax.optimization_barrier(q)
    return q.astype(jnp.float32) * scale


# device time: 1801854 ns/iter; 1.7379x vs baseline; 1.7379x over previous
import jax
import jax.numpy as jnp
from jax import lax
from jax.experimental import pallas as pl
from jax.experimental.pallas import tpu as pltpu

NDEV = 16
_MESH = pl.DeviceIdType.MESH


def _body(x_ref, w_ref, out_ref, send0, send1, recv0, recv1, out_sems,
          rs_send, rs_recv, ag_send, ag_recv,
          rs_cr0, rs_cr1, ag_cr0, ag_cr1):
    my = lax.axis_index("i")
    left = lax.rem(my + NDEV - 1, NDEV)
    right = lax.rem(my + 1, NDEV)

    m = x_ref.shape[0]
    n = w_ref.shape[1]
    ch = m // NDEV
    nh = n // 2

    barrier = pltpu.get_barrier_semaphore()
    pl.semaphore_signal(barrier, inc=1, device_id=(left,), device_id_type=_MESH)
    pl.semaphore_signal(barrier, inc=1, device_id=(right,), device_id_type=_MESH)
    pl.semaphore_wait(barrier, 2)

    def cidx(off):
        return lax.rem(my + off + 2 * NDEV, NDEV)

    def chunk_half(c, d):
        return jnp.dot(x_ref[pl.ds(c * ch, ch), :],
                       w_ref[:, pl.ds(d * nh, nh)],
                       precision=lax.Precision.HIGHEST,
                       preferred_element_type=jnp.float32)

    def signal(sem, dev):
        pl.semaphore_signal(sem, inc=1, device_id=(dev,), device_id_type=_MESH)

    send0[0] = chunk_half(cidx(0), 0)
    send1[0] = chunk_half(cidx(0), 1)
    for s in range(NDEV - 1):
        slot = s % 2
        rdma0 = pltpu.make_async_remote_copy(
            src_ref=send0.at[slot], dst_ref=recv0.at[slot],
            send_sem=rs_send.at[0, s], recv_sem=rs_recv.at[0, s],
            device_id=(right,), device_id_type=_MESH)
        rdma1 = pltpu.make_async_remote_copy(
            src_ref=send1.at[slot], dst_ref=recv1.at[slot],
            send_sem=rs_send.at[1, s], recv_sem=rs_recv.at[1, s],
            device_id=(left,), device_id_type=_MESH)
        if s >= 2:
            pl.semaphore_wait(rs_cr0, 1)
            pl.semaphore_wait(rs_cr1, 1)
        rdma0.start()
        rdma1.start()
        nxt0 = chunk_half(cidx(-s - 1), 0)
        nxt1 = chunk_half(cidx(s + 1), 1)
        rdma0.wait()
        rdma1.wait()
        acc0 = nxt0 + recv0[slot]
        acc1 = nxt1 + recv1[slot]
        if s < NDEV - 2:
            send0[1 - slot] = acc0
            send1[1 - slot] = acc1
            if s <= NDEV - 4:
                signal(rs_cr0, left)
                signal(rs_cr1, right)
        else:
            send0[0] = acc0
            send1[0] = acc1
            signal(ag_cr0, left)
            signal(ag_cr1, right)

    own0 = pltpu.make_async_copy(
        send0.at[0], out_ref.at[pl.ds(cidx(1) * ch, ch), pl.ds(0, nh)],
        out_sems.at[0, 0])
    own1 = pltpu.make_async_copy(
        send1.at[0], out_ref.at[pl.ds(cidx(-1) * ch, ch), pl.ds(nh, nh)],
        out_sems.at[1, 0])
    own0.start()
    own1.start()
    own0.wait()
    own1.wait()

    for t in range(NDEV - 1):
        b = t % 2
        src0 = send0.at[0] if t == 0 else recv0.at[1 - b]
        src1 = send1.at[0] if t == 0 else recv1.at[1 - b]
        rdma0 = pltpu.make_async_remote_copy(
            src_ref=src0, dst_ref=recv0.at[b],
            send_sem=ag_send.at[0, t], recv_sem=ag_recv.at[0, t],
            device_id=(right,), device_id_type=_MESH)
        rdma1 = pltpu.make_async_remote_copy(
            src_ref=src1, dst_ref=recv1.at[b],
            send_sem=ag_send.at[1, t], recv_sem=ag_recv.at[1, t],
            device_id=(left,), device_id_type=_MESH)
        if t == 0 or t >= 2:
            pl.semaphore_wait(ag_cr0, 1)
            pl.semaphore_wait(ag_cr1, 1)
        rdma0.start()
        rdma1.start()
        rdma0.wait()
        rdma1.wait()
        cp0 = pltpu.make_async_copy(
            recv0.at[b], out_ref.at[pl.ds(cidx(-t) * ch, ch), pl.ds(0, nh)],
            out_sems.at[0, 1])
        cp1 = pltpu.make_async_copy(
            recv1.at[b], out_ref.at[pl.ds(cidx(t) * ch, ch), pl.ds(nh, nh)],
            out_sems.at[1, 1])
        cp0.start()
        cp1.start()
        cp0.wait()
        cp1.wait()
        if 1 <= t <= NDEV - 3:
            signal(ag_cr0, left)
            signal(ag_cr1, right)


def _gemm_allreduce(x, w):
    m = x.shape[0]
    n = w.shape[1]
    ch = m // NDEV
    nh = n // 2
    return pl.pallas_call(
        _body,
        out_shape=jax.ShapeDtypeStruct((m, n), jnp.float32),
        in_specs=[pl.BlockSpec(memory_space=pltpu.VMEM),
                  pl.BlockSpec(memory_space=pltpu.VMEM)],
        out_specs=pl.BlockSpec(memory_space=pl.ANY),
        scratch_shapes=[
            pltpu.VMEM((2, ch, nh), jnp.float32),
            pltpu.VMEM((2, ch, nh), jnp.float32),
            pltpu.VMEM((2, ch, nh), jnp.float32),
            pltpu.VMEM((2, ch, nh), jnp.float32),
            pltpu.SemaphoreType.DMA((2, 2)),
            pltpu.SemaphoreType.DMA((2, NDEV - 1)),
            pltpu.SemaphoreType.DMA((2, NDEV - 1)),
            pltpu.SemaphoreType.DMA((2, NDEV - 1)),
            pltpu.SemaphoreType.DMA((2, NDEV - 1)),
            pltpu.SemaphoreType.REGULAR,
            pltpu.SemaphoreType.REGULAR,
            pltpu.SemaphoreType.REGULAR,
            pltpu.SemaphoreType.REGULAR,
        ],
        compiler_params=pltpu.CompilerParams(
            collective_id=0,
            vmem_limit_bytes=100 * 1024 * 1024,
        ),
    )(x, w)


def kernel(x, w_mat):
    y = _gemm_allreduce(x, w_mat)
    amax = jnp.max(jnp.abs(y))
    scale = amax / 448.0
    q = (y / scale).astype(jnp.float8_e4m3fn)
    q = lax.optimization_barrier(q)
    return q.astype(jnp.float32) * scale
